# Initial kernel scaffold; baseline (speedup 1.0000x reference)
import jax
import jax.numpy as jnp
from jax.experimental import pallas as pl


def kernel(u, v, entity_table, rel_table, adj_ent, adj_rel, W, b):
    raise NotImplementedError("write your pallas kernel here")


# speedup vs baseline: 2.3739x; 2.3739x over previous
import functools

import jax
import jax.numpy as jnp
from jax import lax
from jax.experimental import pallas as pl
from jax.experimental.pallas import tpu as pltpu
from jax.experimental.pallas import tpu_sc as plsc

NC = 2
NS = 16
NW = NC * NS

DIM = 32
K = 16
NUM_REL = 64
BLK = 128


def _mesh():
    return plsc.VectorSubcoreMesh(core_axis_name="c", subcore_axis_name="s")


def _wid():
    return lax.axis_index("s") * NC + lax.axis_index("c")


def _sc_stage1(u, v, ent, adj_e, adj_r):
    B = u.shape[0]
    n = B // NW

    @functools.partial(
        pl.kernel,
        mesh=_mesh(),
        compiler_params=pltpu.CompilerParams(use_tc_tiling_on_sc=False),
        out_type=(
            jax.ShapeDtypeStruct((B, DIM), jnp.float32),
            jax.ShapeDtypeStruct((B, DIM), jnp.float32),
            jax.ShapeDtypeStruct((B, K), jnp.int32),
            jax.ShapeDtypeStruct((B, K), jnp.int32),
        ),
        scratch_types=[
            pltpu.VMEM((n,), jnp.int32),
            pltpu.VMEM((n,), jnp.int32),
            pltpu.VMEM((n, DIM), jnp.float32),
            pltpu.VMEM((n, DIM), jnp.float32),
            pltpu.VMEM((n, K), jnp.int32),
            pltpu.VMEM((n, K), jnp.int32),
            pltpu.SemaphoreType.DMA,
        ],
    )
    def k(u_h, v_h, e_h, ae_h, ar_h, ue_h, ev_h, ne1_h, nr1_h,
          iu, iv, ue_v, ev_v, ne1_v, nr1_v, sem):
        base = pl.multiple_of(_wid() * n, 8)
        pltpu.sync_copy(u_h.at[pl.ds(base, n)], iu)
        pltpu.sync_copy(v_h.at[pl.ds(base, n)], iv)
        c1 = pltpu.async_copy(e_h.at[iu], ue_v, sem)
        c2 = pltpu.async_copy(e_h.at[iv], ev_v, sem)
        c3 = pltpu.async_copy(ae_h.at[iv], ne1_v, sem)
        c4 = pltpu.async_copy(ar_h.at[iv], nr1_v, sem)
        c1.wait()
        c2.wait()
        c3.wait()
        c4.wait()
        pltpu.sync_copy(ue_v, ue_h.at[pl.ds(base, n)])
        pltpu.sync_copy(ev_v, ev_h.at[pl.ds(base, n)])
        pltpu.sync_copy(ne1_v, ne1_h.at[pl.ds(base, n)])
        pltpu.sync_copy(nr1_v, nr1_h.at[pl.ds(base, n)])

    return k(u, v, ent, adj_e, adj_r)


def _sc_stage2(ne1_flat, ent, adj_e, adj_r):
    N = ne1_flat.shape[0]
    n = N // NW
    C = 1024
    chunks = n // C

    @functools.partial(
        pl.kernel,
        mesh=_mesh(),
        compiler_params=pltpu.CompilerParams(use_tc_tiling_on_sc=False),
        out_type=(
            jax.ShapeDtypeStruct((N, DIM), jnp.float32),
            jax.ShapeDtypeStruct((N, K), jnp.int32),
            jax.ShapeDtypeStruct((N, K), jnp.int32),
        ),
        scratch_types=[
            pltpu.VMEM((C,), jnp.int32),
            pltpu.VMEM((C, DIM), jnp.float32),
            pltpu.VMEM((C, K), jnp.int32),
            pltpu.VMEM((C, K), jnp.int32),
            pltpu.SemaphoreType.DMA,
        ],
    )
    def k(idx_h, e_h, ae_h, ar_h, en1_h, ne2_h, nr2_h,
          idx_v, er_v, ne_v, nr_v, sem):
        tile_base = _wid() * n

        def body(i, carry):
            base = pl.multiple_of(tile_base + i * C, 8)
            pltpu.sync_copy(idx_h.at[pl.ds(base, C)], idx_v)
            c1 = pltpu.async_copy(e_h.at[idx_v], er_v, sem)
            c2 = pltpu.async_copy(ae_h.at[idx_v], ne_v, sem)
            c3 = pltpu.async_copy(ar_h.at[idx_v], nr_v, sem)
            c1.wait()
            c2.wait()
            c3.wait()
            pltpu.sync_copy(er_v, en1_h.at[pl.ds(base, C)])
            pltpu.sync_copy(ne_v, ne2_h.at[pl.ds(base, C)])
            pltpu.sync_copy(nr_v, nr2_h.at[pl.ds(base, C)])
            return carry

        lax.fori_loop(0, chunks, body, 0)

    return k(ne1_flat, ent, adj_e, adj_r)


def _sc_stage3(ne2_flat, ent):
    N = ne2_flat.shape[0]
    n = N // NW
    C = 2048
    chunks = n // C

    @functools.partial(
        pl.kernel,
        mesh=_mesh(),
        compiler_params=pltpu.CompilerParams(use_tc_tiling_on_sc=False),
        out_type=jax.ShapeDtypeStruct((N, DIM), jnp.float32),
        scratch_types=[
            pltpu.VMEM((C,), jnp.int32),
            pltpu.VMEM((C, DIM), jnp.float32),
            pltpu.SemaphoreType.DMA,
        ],
    )
    def k(idx_h, e_h, en2_h, idx_v, er_v, sem):
        tile_base = _wid() * n

        def body(i, carry):
            base = pl.multiple_of(tile_base + i * C, 8)
            pltpu.sync_copy(idx_h.at[pl.ds(base, C)], idx_v)
            pltpu.async_copy(e_h.at[idx_v], er_v, sem).wait()
            pltpu.sync_copy(er_v, en2_h.at[pl.ds(base, C)])
            return carry

        lax.fori_loop(0, chunks, body, 0)

    return k(ne2_flat, ent)


def _lookup(s, nr):
    acc = jnp.broadcast_to(s[:, 0:1], nr.shape)
    for r in range(1, NUM_REL):
        acc = jnp.where(nr == r, s[:, r:r + 1], acc)
    return acc


def _tc_body(ue_r, ev_r, en1_r, en2_r, nr1_r, nr2_r, rt_r, wt_r, b_r, out_r):
    hi = lax.Precision.HIGHEST
    ue = ue_r[...]
    ev = ev_r[...]
    en1 = en1_r[...]
    nr1 = nr1_r[...]
    nr2 = nr2_r[...]
    rt = rt_r[...]
    wt = wt_r[...]
    bb = b_r[...]

    s = jnp.dot(ue, rt, precision=hi)
    sc1 = _lookup(s, nr1)
    sc2 = _lookup(s, nr2)
    p1 = jax.nn.softmax(sc1, axis=-1)
    p2 = jax.nn.softmax(sc2.reshape(BLK, K, K), axis=-1)

    agg1 = jnp.zeros((BLK, DIM), jnp.float32)
    aggt = jnp.zeros((BLK, DIM), jnp.float32)
    for j in range(K):
        seg = en2_r[:, j * K * DIM:(j + 1) * K * DIM]
        w = jnp.broadcast_to(p2[:, j, :, None], (BLK, K, DIM)).reshape(BLK, K * DIM)
        prod = (seg * w).reshape(BLK, K, DIM)
        aggj = jnp.sum(prod, axis=1)
        e1j = en1[:, j * DIM:(j + 1) * DIM]
        h1j = jax.nn.sigmoid(jnp.dot(e1j + aggj, wt, precision=hi) + bb)
        agg1 = agg1 + p1[:, j:j + 1] * e1j
        aggt = aggt + p1[:, j:j + 1] * h1j
    h0 = jax.nn.sigmoid(jnp.dot(ev + agg1, wt, precision=hi) + bb)
    item = jnp.tanh(jnp.dot(h0 + aggt, wt, precision=hi) + bb)
    out = jax.nn.sigmoid(jnp.sum(ue * item, axis=-1))
    out_r[...] = out.reshape(1, 1, BLK)


def _tc_dense(ue, ev, en1r, en2r, nr1, nr2r, rt, wt, b2):
    B = ue.shape[0]
    G = B // BLK
    out = pl.pallas_call(
        _tc_body,
        grid=(G,),
        in_specs=[
            pl.BlockSpec((BLK, DIM), lambda i: (i, 0)),
            pl.BlockSpec((BLK, DIM), lambda i: (i, 0)),
            pl.BlockSpec((BLK, K * DIM), lambda i: (i, 0)),
            pl.BlockSpec((BLK, K * K * DIM), lambda i: (i, 0)),
            pl.BlockSpec((BLK, K), lambda i: (i, 0)),
            pl.BlockSpec((BLK, K * K), lambda i: (i, 0)),
            pl.BlockSpec((DIM, NUM_REL), lambda i: (0, 0)),
            pl.BlockSpec((DIM, DIM), lambda i: (0, 0)),
            pl.BlockSpec((1, DIM), lambda i: (0, 0)),
        ],
        out_specs=pl.BlockSpec((1, 1, BLK), lambda i: (i, 0, 0)),
        out_shape=jax.ShapeDtypeStruct((G, 1, BLK), jnp.float32),
    )(ue, ev, en1r, en2r, nr1, nr2r, rt, wt, b2)
    return out.reshape(B)


def kernel(u, v, entity_table, rel_table, adj_ent, adj_rel, W, b):
    B = u.shape[0]
    u = u.astype(jnp.int32)
    v = v.astype(jnp.int32)
    adj_ent = adj_ent.astype(jnp.int32)
    adj_rel = adj_rel.astype(jnp.int32)

    ue, ev, ne1, nr1 = _sc_stage1(u, v, entity_table, adj_ent, adj_rel)
    en1, ne2, nr2 = _sc_stage2(ne1.reshape(B * K), entity_table, adj_ent, adj_rel)
    en2 = _sc_stage3(ne2.reshape(B * K * K), entity_table)

    en1r = en1.reshape(B, K * DIM)
    en2r = en2.reshape(B, K * K * DIM)
    nr2r = nr2.reshape(B, K * K)
    rt = rel_table.T
    wt = W.T
    b2 = b.reshape(1, DIM)
    return _tc_dense(ue, ev, en1r, en2r, nr1, nr2r, rt, wt, b2)


# speedup vs baseline: 3.3766x; 1.4224x over previous
import functools

import jax
import jax.numpy as jnp
from jax import lax
from jax.experimental import pallas as pl
from jax.experimental.pallas import tpu as pltpu
from jax.experimental.pallas import tpu_sc as plsc

NC = 2
NS = 16
NW = NC * NS

DIM = 32
K = 16
NUM_REL = 64
BLK = 512


def _mesh():
    return plsc.VectorSubcoreMesh(core_axis_name="c", subcore_axis_name="s")


def _wid():
    return lax.axis_index("s") * NC + lax.axis_index("c")


def _sc_stage1(u, v, ent, adj_e, adj_r):
    B = u.shape[0]
    n = B // NW

    @functools.partial(
        pl.kernel,
        mesh=_mesh(),
        compiler_params=pltpu.CompilerParams(use_tc_tiling_on_sc=False),
        out_type=(
            jax.ShapeDtypeStruct((B, DIM), jnp.float32),
            jax.ShapeDtypeStruct((B, DIM), jnp.float32),
            jax.ShapeDtypeStruct((B, K), jnp.int32),
            jax.ShapeDtypeStruct((B, K), jnp.int32),
        ),
        scratch_types=[
            pltpu.VMEM((n,), jnp.int32),
            pltpu.VMEM((n,), jnp.int32),
            pltpu.VMEM((n, DIM), jnp.float32),
            pltpu.VMEM((n, DIM), jnp.float32),
            pltpu.VMEM((n, K), jnp.int32),
            pltpu.VMEM((n, K), jnp.int32),
            pltpu.SemaphoreType.DMA,
        ],
    )
    def k(u_h, v_h, e_h, ae_h, ar_h, ue_h, ev_h, ne1_h, nr1_h,
          iu, iv, ue_v, ev_v, ne1_v, nr1_v, sem):
        base = pl.multiple_of(_wid() * n, 8)
        pltpu.sync_copy(u_h.at[pl.ds(base, n)], iu)
        pltpu.sync_copy(v_h.at[pl.ds(base, n)], iv)
        c1 = pltpu.async_copy(e_h.at[iu], ue_v, sem)
        c2 = pltpu.async_copy(e_h.at[iv], ev_v, sem)
        c3 = pltpu.async_copy(ae_h.at[iv], ne1_v, sem)
        c4 = pltpu.async_copy(ar_h.at[iv], nr1_v, sem)
        c1.wait()
        c2.wait()
        c3.wait()
        c4.wait()
        pltpu.sync_copy(ue_v, ue_h.at[pl.ds(base, n)])
        pltpu.sync_copy(ev_v, ev_h.at[pl.ds(base, n)])
        pltpu.sync_copy(ne1_v, ne1_h.at[pl.ds(base, n)])
        pltpu.sync_copy(nr1_v, nr1_h.at[pl.ds(base, n)])

    return k(u, v, ent, adj_e, adj_r)


def _sc_stage2(ne1_flat, ent, adj_e, adj_r):
    N = ne1_flat.shape[0]
    n = N // NW
    C = 1024
    chunks = n // C

    @functools.partial(
        pl.kernel,
        mesh=_mesh(),
        compiler_params=pltpu.CompilerParams(use_tc_tiling_on_sc=False),
        out_type=(
            jax.ShapeDtypeStruct((N, DIM), jnp.float32),
            jax.ShapeDtypeStruct((N, K), jnp.int32),
            jax.ShapeDtypeStruct((N, K), jnp.int32),
        ),
        scratch_types=[
            pltpu.VMEM((C,), jnp.int32),
            pltpu.VMEM((C, DIM), jnp.float32),
            pltpu.VMEM((C, K), jnp.int32),
            pltpu.VMEM((C, K), jnp.int32),
            pltpu.SemaphoreType.DMA,
        ],
    )
    def k(idx_h, e_h, ae_h, ar_h, en1_h, ne2_h, nr2_h,
          idx_v, er_v, ne_v, nr_v, sem):
        tile_base = _wid() * n

        def body(i, carry):
            base = pl.multiple_of(tile_base + i * C, 8)
            pltpu.sync_copy(idx_h.at[pl.ds(base, C)], idx_v)
            c1 = pltpu.async_copy(e_h.at[idx_v], er_v, sem)
            c2 = pltpu.async_copy(ae_h.at[idx_v], ne_v, sem)
            c3 = pltpu.async_copy(ar_h.at[idx_v], nr_v, sem)
            c1.wait()
            c2.wait()
            c3.wait()
            pltpu.sync_copy(er_v, en1_h.at[pl.ds(base, C)])
            pltpu.sync_copy(ne_v, ne2_h.at[pl.ds(base, C)])
            pltpu.sync_copy(nr_v, nr2_h.at[pl.ds(base, C)])
            return carry

        lax.fori_loop(0, chunks, body, 0)

    return k(ne1_flat, ent, adj_e, adj_r)


def _sc_stage3(ne2_flat, nr2_flat, es_flat, ent):
    N = ne2_flat.shape[0]
    NG = N // K
    n_groups = NG // NW
    G = 128
    chunks = n_groups // G
    NB = G // K

    @functools.partial(
        pl.kernel,
        mesh=_mesh(),
        compiler_params=pltpu.CompilerParams(
            use_tc_tiling_on_sc=False, needs_layout_passes=False),
        out_type=jax.ShapeDtypeStruct((NG, DIM), jnp.float32),
        scratch_types=[
            pltpu.VMEM((G * K,), jnp.int32),
            pltpu.VMEM((G * K,), jnp.int32),
            pltpu.VMEM((NB * NUM_REL,), jnp.float32),
            pltpu.VMEM((G * K, DIM), jnp.float32),
            pltpu.VMEM((K,), jnp.float32),
            pltpu.VMEM((G, DIM), jnp.float32),
            pltpu.SemaphoreType.DMA,
        ],
    )
    def k(ne2_h, nr2_h, es_h, e_h, agg_h,
          idx_v, nr_v, es_v, rows_v, p_v, agg_v, sem):
        tile_gbase = _wid() * n_groups

        def chunk_body(i, carry):
            gbase = tile_gbase + i * G
            nbase = pl.multiple_of(gbase * K, 8)
            bbase = pl.multiple_of((gbase // K) * NUM_REL, 8)
            pltpu.sync_copy(ne2_h.at[pl.ds(nbase, G * K)], idx_v)
            pltpu.sync_copy(nr2_h.at[pl.ds(nbase, G * K)], nr_v)
            pltpu.sync_copy(es_h.at[pl.ds(bbase, NB * NUM_REL)], es_v)
            pltpu.async_copy(e_h.at[idx_v], rows_v, sem).wait()

            def group_body(g, carry2):
                rel = nr_v[pl.ds(g * K, K)]
                sc_idx = rel + (g // K) * NUM_REL
                e = plsc.load_gather(es_v, [sc_idx])
                tot = jnp.sum(e)
                p = e / tot
                acc0 = jnp.zeros((K,), jnp.float32)
                acc1 = jnp.zeros((K,), jnp.float32)
                for kk in range(K):
                    pk = p[kk]
                    acc0 = acc0 + pk * rows_v[g * K + kk, 0:K]
                    acc1 = acc1 + pk * rows_v[g * K + kk, K:DIM]
                agg_v[g, 0:K] = acc0
                agg_v[g, K:DIM] = acc1
                return carry2

            lax.fori_loop(0, G, group_body, 0)
            pltpu.sync_copy(agg_v, agg_h.at[pl.ds(pl.multiple_of(gbase, 8), G)])
            return carry

        lax.fori_loop(0, chunks, chunk_body, 0)

    return k(ne2_flat, nr2_flat, es_flat, ent)


def _es_body(ue_r, rt_r, es_r):
    es_r[...] = jnp.exp(jnp.dot(ue_r[...], rt_r[...],
                                precision=lax.Precision.HIGHEST))


def _tc_es(ue, rt):
    B = ue.shape[0]
    return pl.pallas_call(
        _es_body,
        in_specs=[
            pl.BlockSpec((B, DIM), lambda: (0, 0)),
            pl.BlockSpec((DIM, NUM_REL), lambda: (0, 0)),
        ],
        out_specs=pl.BlockSpec((B, NUM_REL), lambda: (0, 0)),
        out_shape=jax.ShapeDtypeStruct((B, NUM_REL), jnp.float32),
    )(ue, rt)


def _lookup(es, nr):
    acc = jnp.broadcast_to(es[:, 0:1], nr.shape)
    for r in range(1, NUM_REL):
        acc = jnp.where(nr == r, es[:, r:r + 1], acc)
    return acc


def _tc_body(ue_r, ev_r, en1_r, agg2_r, nr1_r, es_r, wt_r, b_r, out_r):
    hi = lax.Precision.HIGHEST
    ue = ue_r[...]
    ev = ev_r[...]
    nr1 = nr1_r[...]
    es = es_r[...]
    wt = wt_r[...]
    bb = b_r[...]

    e1 = _lookup(es, nr1)
    p1 = e1 / jnp.sum(e1, axis=-1, keepdims=True)

    agg1 = jnp.zeros((BLK, DIM), jnp.float32)
    aggt = jnp.zeros((BLK, DIM), jnp.float32)
    for j in range(K):
        e1j = en1_r[:, j * DIM:(j + 1) * DIM]
        a2j = agg2_r[:, j * DIM:(j + 1) * DIM]
        h1j = jax.nn.sigmoid(jnp.dot(e1j + a2j, wt, precision=hi) + bb)
        agg1 = agg1 + p1[:, j:j + 1] * e1j
        aggt = aggt + p1[:, j:j + 1] * h1j
    h0 = jax.nn.sigmoid(jnp.dot(ev + agg1, wt, precision=hi) + bb)
    item = jnp.tanh(jnp.dot(h0 + aggt, wt, precision=hi) + bb)
    out = jax.nn.sigmoid(jnp.sum(ue * item, axis=-1))
    out_r[...] = out.reshape(1, 1, BLK)


def _tc_dense(ue, ev, en1r, agg2r, nr1, es, wt, b2):
    B = ue.shape[0]
    Gn = B // BLK
    out = pl.pallas_call(
        _tc_body,
        grid=(Gn,),
        in_specs=[
            pl.BlockSpec((BLK, DIM), lambda i: (i, 0)),
            pl.BlockSpec((BLK, DIM), lambda i: (i, 0)),
            pl.BlockSpec((BLK, K * DIM), lambda i: (i, 0)),
            pl.BlockSpec((BLK, K * DIM), lambda i: (i, 0)),
            pl.BlockSpec((BLK, K), lambda i: (i, 0)),
            pl.BlockSpec((BLK, NUM_REL), lambda i: (i, 0)),
            pl.BlockSpec((DIM, DIM), lambda i: (0, 0)),
            pl.BlockSpec((1, DIM), lambda i: (0, 0)),
        ],
        out_specs=pl.BlockSpec((1, 1, BLK), lambda i: (i, 0, 0)),
        out_shape=jax.ShapeDtypeStruct((Gn, 1, BLK), jnp.float32),
    )(ue, ev, en1r, agg2r, nr1, es, wt, b2)
    return out.reshape(B)


def kernel(u, v, entity_table, rel_table, adj_ent, adj_rel, W, b):
    B = u.shape[0]
    u = u.astype(jnp.int32)
    v = v.astype(jnp.int32)
    adj_ent = adj_ent.astype(jnp.int32)
    adj_rel = adj_rel.astype(jnp.int32)

    ue, ev, ne1, nr1 = _sc_stage1(u, v, entity_table, adj_ent, adj_rel)
    es = _tc_es(ue, rel_table.T)
    en1, ne2, nr2 = _sc_stage2(ne1.reshape(B * K), entity_table, adj_ent, adj_rel)
    agg2 = _sc_stage3(ne2.reshape(B * K * K), nr2.reshape(B * K * K),
                      es.reshape(B * NUM_REL), entity_table)

    en1r = en1.reshape(B, K * DIM)
    agg2r = agg2.reshape(B, K * DIM)
    wt = W.T
    b2 = b.reshape(1, DIM)
    return _tc_dense(ue, ev, en1r, agg2r, nr1, es, wt, b2)


# speedup vs baseline: 11.2000x; 3.3169x over previous
import functools

import jax
import jax.numpy as jnp
from jax import lax
from jax.experimental import pallas as pl
from jax.experimental.pallas import tpu as pltpu
from jax.experimental.pallas import tpu_sc as plsc

NC = 2
NS = 16
NW = NC * NS

DIM = 32
K = 16
NUM_REL = 64
BLK = 512


def _mesh():
    return plsc.VectorSubcoreMesh(core_axis_name="c", subcore_axis_name="s")


def _wid():
    return lax.axis_index("s") * NC + lax.axis_index("c")


_SC_PARAMS = pltpu.CompilerParams(
    use_tc_tiling_on_sc=False, needs_layout_passes=False)


def _remap_row(x, pack_log):
    m = (1 << pack_log) - 1
    blk = x & (-(1024 << pack_log))
    return blk | ((x & 1023) << pack_log) | ((x >> 10) & m)


def _remap_idx(src_v, dst_v, n, pack_log):
    def body(t, c):
        x = src_v[pl.ds(t * 16, 16)]
        dst_v[pl.ds(t * 16, 16)] = _remap_row(x, pack_log)
        return c

    lax.fori_loop(0, n // 16, body, 0)


_EC = 16384
_AC = 16384


def _tr_e_body(x_ref, o_ref):
    x = x_ref[...]
    for g in range(_EC // 4096):
        stk = jnp.concatenate(
            [x[:, g * 4096 + q * 1024:g * 4096 + (q + 1) * 1024]
             for q in range(4)], axis=0)
        o_ref[g * 1024:(g + 1) * 1024, :] = stk.T


def _tr_a_body(a_ref, r_ref, oa_ref, or_ref):
    a = a_ref[...]
    r = r_ref[...]
    for g in range(_AC // 8192):
        sa = jnp.concatenate(
            [a[:, g * 8192 + q * 1024:g * 8192 + (q + 1) * 1024]
             for q in range(8)], axis=0)
        sr = jnp.concatenate(
            [r[:, g * 8192 + q * 1024:g * 8192 + (q + 1) * 1024]
             for q in range(8)], axis=0)
        oa_ref[g * 1024:(g + 1) * 1024, :] = sa.T
        or_ref[g * 1024:(g + 1) * 1024, :] = sr.T


def _relayout_tables(ent, adj_e, adj_r):
    T = ent.shape[0]
    ge = -(-T // _EC)
    epad = pl.pallas_call(
        _tr_e_body,
        grid=(ge,),
        in_specs=[pl.BlockSpec((DIM, _EC), lambda i: (0, i))],
        out_specs=pl.BlockSpec((_EC // 4, 128), lambda i: (i, 0)),
        out_shape=jax.ShapeDtypeStruct((ge * _EC // 4, 128), jnp.float32),
    )(ent.T)
    e_rm = epad.reshape(ge * _EC, DIM)

    ga = -(-T // _AC)
    apad, rpad = pl.pallas_call(
        _tr_a_body,
        grid=(ga,),
        in_specs=[
            pl.BlockSpec((K, _AC), lambda i: (0, i)),
            pl.BlockSpec((K, _AC), lambda i: (0, i)),
        ],
        out_specs=[
            pl.BlockSpec((_AC // 8, 128), lambda i: (i, 0)),
            pl.BlockSpec((_AC // 8, 128), lambda i: (i, 0)),
        ],
        out_shape=[
            jax.ShapeDtypeStruct((ga * _AC // 8, 128), jnp.int32),
            jax.ShapeDtypeStruct((ga * _AC // 8, 128), jnp.int32),
        ],
    )(adj_e.T, adj_r.T)
    ae_rm = apad.reshape(ga * _AC, K)
    ar_rm = rpad.reshape(ga * _AC, K)
    return e_rm, ae_rm, ar_rm


def _sc_stage1(u, v, ent, adj_e, adj_r):
    B = u.shape[0]
    n = B // NW

    @functools.partial(
        pl.kernel,
        mesh=_mesh(),
        compiler_params=_SC_PARAMS,
        out_type=(
            jax.ShapeDtypeStruct((B, DIM), jnp.float32),
            jax.ShapeDtypeStruct((B, DIM), jnp.float32),
            jax.ShapeDtypeStruct((B, K), jnp.int32),
            jax.ShapeDtypeStruct((B, K), jnp.int32),
        ),
        scratch_types=[
            pltpu.VMEM((n,), jnp.int32),
            pltpu.VMEM((n,), jnp.int32),
            pltpu.VMEM((n,), jnp.int32),
            pltpu.VMEM((n,), jnp.int32),
            pltpu.VMEM((n,), jnp.int32),
            pltpu.VMEM((n, DIM), jnp.float32),
            pltpu.VMEM((n, DIM), jnp.float32),
            pltpu.VMEM((n, K), jnp.int32),
            pltpu.VMEM((n, K), jnp.int32),
            pltpu.SemaphoreType.DMA,
        ],
    )
    def k(u_h, v_h, e_h, ae_h, ar_h, ue_h, ev_h, ne1_h, nr1_h,
          iu, iv, ifu, ifv, igv, ue_v, ev_v, ne1_v, nr1_v, sem):
        base = pl.multiple_of(_wid() * n, 8)
        pltpu.sync_copy(u_h.at[pl.ds(base, n)], iu)
        pltpu.sync_copy(v_h.at[pl.ds(base, n)], iv)
        _remap_idx(iu, ifu, n, 2)
        _remap_idx(iv, ifv, n, 2)
        _remap_idx(iv, igv, n, 3)
        c1 = pltpu.async_copy(e_h.at[ifu], ue_v, sem)
        c2 = pltpu.async_copy(e_h.at[ifv], ev_v, sem)
        c3 = pltpu.async_copy(ae_h.at[igv], ne1_v, sem)
        c4 = pltpu.async_copy(ar_h.at[igv], nr1_v, sem)
        c1.wait()
        c2.wait()
        c3.wait()
        c4.wait()
        pltpu.sync_copy(ue_v, ue_h.at[pl.ds(base, n)])
        pltpu.sync_copy(ev_v, ev_h.at[pl.ds(base, n)])
        pltpu.sync_copy(ne1_v, ne1_h.at[pl.ds(base, n)])
        pltpu.sync_copy(nr1_v, nr1_h.at[pl.ds(base, n)])

    return k(u, v, ent, adj_e, adj_r)


def _sc_stage2(ne1_flat, ent, adj_e, adj_r):
    N = ne1_flat.shape[0]
    n = N // NW
    C = 1024
    chunks = n // C

    @functools.partial(
        pl.kernel,
        mesh=_mesh(),
        compiler_params=_SC_PARAMS,
        out_type=(
            jax.ShapeDtypeStruct((N, DIM), jnp.float32),
            jax.ShapeDtypeStruct((N, K), jnp.int32),
            jax.ShapeDtypeStruct((N, K), jnp.int32),
        ),
        scratch_types=[
            pltpu.VMEM((C,), jnp.int32),
            pltpu.VMEM((C,), jnp.int32),
            pltpu.VMEM((C,), jnp.int32),
            pltpu.VMEM((C, DIM), jnp.float32),
            pltpu.VMEM((C, K), jnp.int32),
            pltpu.VMEM((C, K), jnp.int32),
            pltpu.SemaphoreType.DMA,
        ],
    )
    def k(idx_h, e_h, ae_h, ar_h, en1_h, ne2_h, nr2_h,
          idx_v, idf_v, idg_v, er_v, ne_v, nr_v, sem):
        tile_base = _wid() * n

        def body(i, carry):
            base = pl.multiple_of(tile_base + i * C, 8)
            pltpu.sync_copy(idx_h.at[pl.ds(base, C)], idx_v)
            _remap_idx(idx_v, idf_v, C, 2)
            _remap_idx(idx_v, idg_v, C, 3)
            c1 = pltpu.async_copy(e_h.at[idf_v], er_v, sem)
            c2 = pltpu.async_copy(ae_h.at[idg_v], ne_v, sem)
            c3 = pltpu.async_copy(ar_h.at[idg_v], nr_v, sem)
            c1.wait()
            c2.wait()
            c3.wait()
            pltpu.sync_copy(er_v, en1_h.at[pl.ds(base, C)])
            pltpu.sync_copy(ne_v, ne2_h.at[pl.ds(base, C)])
            pltpu.sync_copy(nr_v, nr2_h.at[pl.ds(base, C)])
            return carry

        lax.fori_loop(0, chunks, body, 0)

    return k(ne1_flat, ent, adj_e, adj_r)


def _sc_stage3(ne2, nr2, nr1, es_flat, ent):
    NG = ne2.shape[0]
    B = NG // K
    n_groups = NG // NW
    BT = B // NW
    G = 64
    chunks = n_groups // G
    pairs = chunks // 2

    @functools.partial(
        pl.kernel,
        mesh=_mesh(),
        compiler_params=_SC_PARAMS,
        out_type=(
            jax.ShapeDtypeStruct((NG, DIM), jnp.float32),
            jax.ShapeDtypeStruct((B, K), jnp.float32),
        ),
        scratch_types=[
            pltpu.VMEM((BT * NUM_REL,), jnp.float32),
            pltpu.VMEM((BT, K), jnp.int32),
            pltpu.VMEM((BT, K), jnp.float32),
            pltpu.VMEM((G, K), jnp.int32),
            pltpu.VMEM((G, K), jnp.int32),
            pltpu.VMEM((G * K,), jnp.int32),
            pltpu.VMEM((G * K,), jnp.int32),
            pltpu.VMEM((G, K), jnp.int32),
            pltpu.VMEM((G, K), jnp.int32),
            pltpu.VMEM((G * K, DIM), jnp.float32),
            pltpu.VMEM((G * K, DIM), jnp.float32),
            pltpu.VMEM((G, DIM), jnp.float32),
            pltpu.SemaphoreType.DMA,
            pltpu.SemaphoreType.DMA,
        ],
    )
    def k(ne2_h, nr2_h, nr1_h, es_h, e_h, agg_h, p1_h,
          es_v, nr1_v, p1_v, ixa, ixb, ifa, ifb, nra, nrb,
          rwa, rwb, agg_v, sema, semb):
        wid = _wid()
        tb = pl.multiple_of(wid * BT, 8)
        tile_gbase = wid * n_groups
        pltpu.sync_copy(es_h.at[pl.ds(tb * NUM_REL, BT * NUM_REL)], es_v)
        pltpu.sync_copy(nr1_h.at[pl.ds(tb, BT)], nr1_v)

        def p1_body(b, c):
            rel = nr1_v[b, :]
            e = plsc.load_gather(es_v, [rel + b * NUM_REL])
            p1_v[b, :] = e / jnp.sum(e)
            return c

        lax.fori_loop(0, BT, p1_body, 0)
        pltpu.sync_copy(p1_v, p1_h.at[pl.ds(tb, BT)])

        def stage(c, ix_v, if_v, nr_v, rw_v, sem):
            gb = pl.multiple_of(tile_gbase + c * G, 8)
            pltpu.sync_copy(ne2_h.at[pl.ds(gb, G)], ix_v)
            pltpu.sync_copy(nr2_h.at[pl.ds(gb, G)], nr_v)

            def rbody(t, cc):
                if_v[pl.ds(t * K, K)] = _remap_row(ix_v[t, :], 2)
                return cc

            lax.fori_loop(0, G, rbody, 0)
            pltpu.async_copy(e_h.at[if_v], rw_v, sem)

        def compute(c, nr_v, rw_v, sem):
            pltpu.make_async_copy(e_h.at[pl.ds(0, G * K)], rw_v, sem).wait()

            def group_body(g, cc):
                b_loc = (c * G + g) // K
                rel = nr_v[g, :]
                e = plsc.load_gather(es_v, [rel + b_loc * NUM_REL])
                p = e / jnp.sum(e)
                acc0 = jnp.zeros((K,), jnp.float32)
                acc1 = jnp.zeros((K,), jnp.float32)
                for kk in range(K):
                    pk = p[kk]
                    acc0 = acc0 + pk * rw_v[g * K + kk, 0:K]
                    acc1 = acc1 + pk * rw_v[g * K + kk, K:DIM]
                agg_v[g, 0:K] = acc0
                agg_v[g, K:DIM] = acc1
                return cc

            lax.fori_loop(0, G, group_body, 0)
            pltpu.sync_copy(
                agg_v, agg_h.at[pl.ds(pl.multiple_of(tile_gbase + c * G, 8), G)])

        stage(0, ixa, ifa, nra, rwa, sema)

        def pair_body(i, carry):
            stage(2 * i + 1, ixb, ifb, nrb, rwb, semb)
            compute(2 * i, nra, rwa, sema)

            @pl.when(i < pairs - 1)
            def _():
                stage(2 * i + 2, ixa, ifa, nra, rwa, sema)

            compute(2 * i + 1, nrb, rwb, semb)
            return carry

        lax.fori_loop(0, pairs, pair_body, 0)

    return k(ne2, nr2, nr1, es_flat, ent)


def _es_body(ue_r, rt_r, es_r):
    es_r[...] = jnp.exp(jnp.dot(ue_r[...], rt_r[...],
                                precision=lax.Precision.HIGHEST))


def _tc_es(ue, rt):
    B = ue.shape[0]
    return pl.pallas_call(
        _es_body,
        in_specs=[
            pl.BlockSpec((B, DIM), lambda: (0, 0)),
            pl.BlockSpec((DIM, NUM_REL), lambda: (0, 0)),
        ],
        out_specs=pl.BlockSpec((B, NUM_REL), lambda: (0, 0)),
        out_shape=jax.ShapeDtypeStruct((B, NUM_REL), jnp.float32),
    )(ue, rt)


def _tc_body(ue_r, ev_r, en1_r, agg2_r, p1_r, wbd_r, wt_r, b_r, bt_r, out_r):
    hi = lax.Precision.HIGHEST
    ue = ue_r[...]
    ev = ev_r[...]
    en1 = en1_r[...]
    p1 = p1_r[...]
    wt = wt_r[...]
    bb = b_r[...]

    x = en1 + agg2_r[...]
    h1 = jax.nn.sigmoid(
        jnp.dot(x, wbd_r[...], precision=hi) + bt_r[...])

    agg1 = jnp.zeros((BLK, DIM), jnp.float32)
    aggt = jnp.zeros((BLK, DIM), jnp.float32)
    for j in range(K):
        pj = p1[:, j:j + 1]
        agg1 = agg1 + pj * en1[:, j * DIM:(j + 1) * DIM]
        aggt = aggt + pj * h1[:, j * DIM:(j + 1) * DIM]
    h0 = jax.nn.sigmoid(jnp.dot(ev + agg1, wt, precision=hi) + bb)
    item = jnp.tanh(jnp.dot(h0 + aggt, wt, precision=hi) + bb)
    out = jax.nn.sigmoid(jnp.sum(ue * item, axis=-1))
    out_r[...] = out.reshape(1, 1, BLK)


def _tc_dense(ue, ev, en1r, agg2r, p1, wbd, wt, b2, bt):
    B = ue.shape[0]
    Gn = B // BLK
    out = pl.pallas_call(
        _tc_body,
        grid=(Gn,),
        in_specs=[
            pl.BlockSpec((BLK, DIM), lambda i: (i, 0)),
            pl.BlockSpec((BLK, DIM), lambda i: (i, 0)),
            pl.BlockSpec((BLK, K * DIM), lambda i: (i, 0)),
            pl.BlockSpec((BLK, K * DIM), lambda i: (i, 0)),
            pl.BlockSpec((BLK, K), lambda i: (i, 0)),
            pl.BlockSpec((K * DIM, K * DIM), lambda i: (0, 0)),
            pl.BlockSpec((DIM, DIM), lambda i: (0, 0)),
            pl.BlockSpec((1, DIM), lambda i: (0, 0)),
            pl.BlockSpec((1, K * DIM), lambda i: (0, 0)),
        ],
        out_specs=pl.BlockSpec((1, 1, BLK), lambda i: (i, 0, 0)),
        out_shape=jax.ShapeDtypeStruct((Gn, 1, BLK), jnp.float32),
    )(ue, ev, en1r, agg2r, p1, wbd, wt, b2, bt)
    return out.reshape(B)


def kernel(u, v, entity_table, rel_table, adj_ent, adj_rel, W, b):
    B = u.shape[0]
    u = u.astype(jnp.int32)
    v = v.astype(jnp.int32)
    adj_ent = adj_ent.astype(jnp.int32)
    adj_rel = adj_rel.astype(jnp.int32)

    e_rm, ae_rm, ar_rm = _relayout_tables(entity_table, adj_ent, adj_rel)
    ue, ev, ne1, nr1 = _sc_stage1(u, v, e_rm, ae_rm, ar_rm)
    es = _tc_es(ue, rel_table.T)
    en1, ne2, nr2 = _sc_stage2(ne1.reshape(B * K), e_rm, ae_rm, ar_rm)
    agg2, p1 = _sc_stage3(ne2, nr2, nr1, es.reshape(B * NUM_REL), e_rm)

    en1r = en1.reshape(B, K * DIM)
    agg2r = agg2.reshape(B, K * DIM)
    wt = W.T
    wbd = jnp.kron(jnp.eye(K, dtype=jnp.float32), wt)
    b2 = b.reshape(1, DIM)
    bt = jnp.tile(b, K).reshape(1, K * DIM)
    return _tc_dense(ue, ev, en1r, agg2r, p1, wbd, wt, b2, bt)
